# baseline (speedup 1.0000x reference)
import functools
import jax
import jax.numpy as jnp
from jax.experimental import pallas as pl
from jax.experimental.pallas import tpu as pltpu

N = 50000
E = 800000
D = 100
MSG = 100
TDIM = 100
H = 2
C = D // H
EDGE_DIM = MSG + TDIM

BN = 2000
BE = 6400


def _qkvs_body(x_ref, w_ref, b_ref, o_ref):
    x = x_ref[...]
    o_ref[...] = jnp.dot(x, w_ref[...], preferred_element_type=jnp.float32) + b_ref[...]


def _project_qkvs(x, W4, b4):
    grid = (N // BN,)
    return pl.pallas_call(
        _qkvs_body,
        grid=grid,
        in_specs=[
            pl.BlockSpec((BN, D), lambda i: (i, 0)),
            pl.BlockSpec((D, 4 * D), lambda i: (0, 0)),
            pl.BlockSpec((1, 4 * D), lambda i: (0, 0)),
        ],
        out_specs=pl.BlockSpec((BN, 4 * D), lambda i: (i, 0)),
        out_shape=jax.ShapeDtypeStruct((N, 4 * D), jnp.float32),
    )(x, W4, b4)


def _edge_body(lu_ref, t_ref, msg_ref, wt_ref, bt_ref, we_ref, o_ref):
    rel = lu_ref[...] - t_ref[...]
    enc = jnp.cos(rel.reshape(BE, 1) * wt_ref[...] + bt_ref[...])
    ea = jnp.concatenate([enc, msg_ref[...]], axis=1)
    o_ref[...] = jnp.dot(ea, we_ref[...], preferred_element_type=jnp.float32)


def _edge_e(lu_src, t, msg, Wt, bt, We):
    grid = (E // BE,)
    return pl.pallas_call(
        _edge_body,
        grid=grid,
        in_specs=[
            pl.BlockSpec((1, 1, BE), lambda i: (i, 0, 0)),
            pl.BlockSpec((1, 1, BE), lambda i: (i, 0, 0)),
            pl.BlockSpec((BE, MSG), lambda i: (i, 0)),
            pl.BlockSpec((1, TDIM), lambda i: (0, 0)),
            pl.BlockSpec((1, TDIM), lambda i: (0, 0)),
            pl.BlockSpec((EDGE_DIM, D), lambda i: (0, 0)),
        ],
        out_specs=pl.BlockSpec((BE, D), lambda i: (i, 0)),
        out_shape=jax.ShapeDtypeStruct((E, D), jnp.float32),
    )(lu_src.reshape(E // BE, 1, BE), t.reshape(E // BE, 1, BE), msg, Wt, bt, We)


def kernel(x, last_update, edge_index, t, msg, Wq, bq, Wk, bk, Wv, bv, We, Wskip, bskip, Wt, bt):
    src = edge_index[0].astype(jnp.int32)
    dst = edge_index[1].astype(jnp.int32)

    W4 = jnp.concatenate([Wq, Wk, Wv, Wskip], axis=1)
    b4 = jnp.concatenate([bq, bk, bv, bskip]).reshape(1, 4 * D)
    qkvs = _project_qkvs(x, W4, b4)
    q = qkvs[:, 0:D]
    k = qkvs[:, D:2 * D]
    v = qkvs[:, 2 * D:3 * D]
    skip = qkvs[:, 3 * D:4 * D]

    lu_src = last_update[src]
    e = _edge_e(lu_src, t, msg, Wt.reshape(1, TDIM), bt.reshape(1, TDIM), We)

    q_i = q[dst].reshape(-1, H, C)
    k_j = (k[src] + e).reshape(-1, H, C)
    v_j = (v[src] + e).reshape(-1, H, C)
    alpha = jnp.sum(q_i * k_j, axis=-1) / jnp.sqrt(jnp.float32(C))
    amax = jax.ops.segment_max(alpha, dst, num_segments=N)
    amax = jnp.where(jnp.isfinite(amax), amax, 0.0)
    ex = jnp.exp(alpha - amax[dst])
    denom = jax.ops.segment_sum(ex, dst, num_segments=N)
    attn = ex / (denom[dst] + 1e-16)
    m = attn[:, :, None] * v_j
    out = jax.ops.segment_sum(m, dst, num_segments=N)
    out = out.reshape(N, H * C)
    return out + skip


# speedup vs baseline: 15.3210x; 15.3210x over previous
import functools
import jax
import jax.numpy as jnp
from jax import lax
from jax.experimental import pallas as pl
from jax.experimental.pallas import tpu as pltpu
from jax.experimental.pallas import tpu_sc as plsc

N = 50000
E = 800000
D = 100
MSG = 100
TDIM = 100
H = 2
C = D // H
EDGE_DIM = MSG + TDIM

BN = 2000
BE = 6400

NC = 2
NS = 16
NW = NC * NS
EPW = E // NW
W = 128
FW = EPW // W
REM = EPW - FW * W

EPT3 = E // NS
FW3 = EPT3 // W
REM3 = EPT3 - FW3 * W
RNG = 12504
NJUNK = 64
DPAD = 50176
DSL = DPAD // NS

_mesh = plsc.VectorSubcoreMesh(
    core_axis_name="c", subcore_axis_name="s", num_cores=NC, num_subcores=NS)

_f32 = jnp.float32
_i32 = jnp.int32


def _iota16():
    return lax.iota(_i32, 16)


DP = 128


def _qkvs_body(x_ref, w_ref, b_ref, o_q, o_k, o_v, o_s):
    r = jnp.dot(x_ref[...], w_ref[...], preferred_element_type=_f32) + b_ref[...]
    o_q[...] = r[:, 0:DP]
    o_k[...] = r[:, DP:2 * DP]
    o_v[...] = r[:, 2 * DP:3 * DP]
    o_s[...] = r[:, 3 * DP:4 * DP]


def _project_qkvs(x, W4, b4):
    return pl.pallas_call(
        _qkvs_body,
        grid=(N // BN,),
        in_specs=[
            pl.BlockSpec((BN, D), lambda i: (i, 0)),
            pl.BlockSpec((D, 4 * DP), lambda i: (0, 0)),
            pl.BlockSpec((1, 4 * DP), lambda i: (0, 0)),
        ],
        out_specs=[pl.BlockSpec((BN, DP), lambda i: (i, 0))] * 4,
        out_shape=[jax.ShapeDtypeStruct((N, DP), _f32)] * 4,
    )(x, W4, b4)


@functools.partial(
    pl.kernel,
    out_type=(
        jax.ShapeDtypeStruct((E,), _f32),
        jax.ShapeDtypeStruct((E, DP), _f32),
        jax.ShapeDtypeStruct((E, DP), _f32),
        jax.ShapeDtypeStruct((E, DP), _f32),
    ),
    mesh=_mesh,
    scratch_types=[
        pltpu.VMEM((W,), _i32),
        pltpu.VMEM((W,), _i32),
        pltpu.VMEM((W,), _f32),
        pltpu.VMEM((W, DP), _f32),
        pltpu.VMEM((W, DP), _f32),
        pltpu.VMEM((W, DP), _f32),
        pltpu.SemaphoreType.DMA,
    ],
)
def _sc_gather1(lastu, qh, kh, vh, srch, dsth, lu_o, qd_o, ks_o, vs_o,
                sbuf, dbuf, lubuf, qbuf, kbuf, vbuf, sem):
    wid = lax.axis_index("c") * NS + lax.axis_index("s")
    base0 = wid * EPW

    def do_window(base, n):
        pltpu.sync_copy(srch.at[pl.ds(base, n)], sbuf.at[pl.ds(0, n)])
        pltpu.sync_copy(dsth.at[pl.ds(base, n)], dbuf.at[pl.ds(0, n)])
        c1 = pltpu.async_copy(lastu.at[sbuf], lubuf, sem)
        c2 = pltpu.async_copy(qh.at[dbuf], qbuf, sem)
        c3 = pltpu.async_copy(kh.at[sbuf], kbuf, sem)
        c4 = pltpu.async_copy(vh.at[sbuf], vbuf, sem)
        c1.wait(); c2.wait(); c3.wait(); c4.wait()
        pltpu.sync_copy(lubuf.at[pl.ds(0, n)], lu_o.at[pl.ds(base, n)])
        pltpu.sync_copy(qbuf.at[pl.ds(0, n)], qd_o.at[pl.ds(base, n)])
        pltpu.sync_copy(kbuf.at[pl.ds(0, n)], ks_o.at[pl.ds(base, n)])
        pltpu.sync_copy(vbuf.at[pl.ds(0, n)], vs_o.at[pl.ds(base, n)])

    def body(i, carry):
        do_window(base0 + i * W, W)
        return carry

    lax.fori_loop(0, FW, body, 0)
    zero = jnp.zeros((16,), _i32)
    for j in range(W // 16):
        sbuf[pl.ds(j * 16, 16)] = zero
        dbuf[pl.ds(j * 16, 16)] = zero
    do_window(base0 + FW * W, REM)


def _edge_body(lu_ref, t_ref, msg_ref, qd_ref, ks_ref, wt_ref, bt_ref, we_ref,
               e_ref, a0_ref, a1_ref, m_ref):
    i = pl.program_id(0)
    rel = lu_ref[...] - t_ref[...]
    enc = jnp.cos(rel.reshape(BE, 1) * wt_ref[...] + bt_ref[...])
    ea = jnp.concatenate([enc, msg_ref[...]], axis=1)
    e = jnp.dot(ea, we_ref[...], preferred_element_type=_f32)
    e_ref[...] = e
    prod = qd_ref[:, 0:D] * (ks_ref[:, 0:D] + e)
    col = lax.broadcasted_iota(_i32, (BE, D), 1)
    inv = 1.0 / jnp.sqrt(jnp.float32(C))
    a0 = jnp.sum(jnp.where(col < C, prod, 0.0), axis=1) * inv
    a1 = jnp.sum(jnp.where(col >= C, prod, 0.0), axis=1) * inv
    a0_ref[...] = a0.reshape(1, 1, BE)
    a1_ref[...] = a1.reshape(1, 1, BE)
    bm = jnp.maximum(jnp.max(a0), jnp.max(a1))

    @pl.when(i == 0)
    def _():
        m_ref[...] = jnp.full((8, 128), -jnp.inf, _f32)

    m_ref[...] = jnp.maximum(m_ref[...], bm)


def _edge_alpha(lu_src, t, msg, qd, ks, Wt, bt, We):
    nb = E // BE
    return pl.pallas_call(
        _edge_body,
        grid=(nb,),
        in_specs=[
            pl.BlockSpec((1, 1, BE), lambda i: (i, 0, 0)),
            pl.BlockSpec((1, 1, BE), lambda i: (i, 0, 0)),
            pl.BlockSpec((BE, MSG), lambda i: (i, 0)),
            pl.BlockSpec((BE, DP), lambda i: (i, 0)),
            pl.BlockSpec((BE, DP), lambda i: (i, 0)),
            pl.BlockSpec((1, TDIM), lambda i: (0, 0)),
            pl.BlockSpec((1, TDIM), lambda i: (0, 0)),
            pl.BlockSpec((EDGE_DIM, D), lambda i: (0, 0)),
        ],
        out_specs=[
            pl.BlockSpec((BE, D), lambda i: (i, 0)),
            pl.BlockSpec((1, 1, BE), lambda i: (i, 0, 0)),
            pl.BlockSpec((1, 1, BE), lambda i: (i, 0, 0)),
            pl.BlockSpec((8, 128), lambda i: (0, 0)),
        ],
        out_shape=[
            jax.ShapeDtypeStruct((E, D), _f32),
            jax.ShapeDtypeStruct((nb, 1, BE), _f32),
            jax.ShapeDtypeStruct((nb, 1, BE), _f32),
            jax.ShapeDtypeStruct((8, 128), _f32),
        ],
    )(lu_src.reshape(nb, 1, BE), t.reshape(nb, 1, BE), msg, qd, ks,
      Wt.reshape(1, TDIM), bt.reshape(1, TDIM), We)


@functools.partial(
    pl.kernel,
    out_type=jax.ShapeDtypeStruct((NC * H * DPAD,), _f32),
    mesh=_mesh,
    scratch_types=[
        pltpu.VMEM((DSL,), _f32),
        pltpu.VMEM((W,), _i32),
        pltpu.VMEM((W,), _f32),
        pltpu.VMEM((W,), _f32),
        pltpu.VMEM((W,), _f32),
        pltpu.VMEM((W,), _f32),
        pltpu.VMEM((16,), _f32),
        pltpu.VMEM_SHARED((DPAD,), _f32),
        pltpu.VMEM_SHARED((DPAD,), _f32),
    ],
)
def _sc_denom(a0h, a1h, dsth, mh, dp_o,
              zbuf, dbuf, a0buf, a1buf, e0buf, e1buf, mbuf, d0s, d1s):
    cid = lax.axis_index("c")
    sid = lax.axis_index("s")
    wid = cid * NS + sid
    base0 = wid * EPW

    zero = jnp.zeros((16,), _f32)

    def zb(i, carry):
        zbuf[pl.ds(i * 16, 16)] = zero
        return carry

    lax.fori_loop(0, DSL // 16, zb, 0)
    pltpu.sync_copy(zbuf, d0s.at[pl.ds(sid * DSL, DSL)])
    pltpu.sync_copy(zbuf, d1s.at[pl.ds(sid * DSL, DSL)])
    pltpu.sync_copy(mh, mbuf)
    plsc.subcore_barrier()

    msplat = mbuf[...]
    iota = _iota16()

    def do_window(base, n):
        pltpu.sync_copy(dsth.at[pl.ds(base, n)], dbuf.at[pl.ds(0, n)])
        pltpu.sync_copy(a0h.at[pl.ds(base, n)], a0buf.at[pl.ds(0, n)])
        pltpu.sync_copy(a1h.at[pl.ds(base, n)], a1buf.at[pl.ds(0, n)])
        for j in range(W // 16):
            valid = (j * 16 + iota) < n
            a0v = a0buf[pl.ds(j * 16, 16)]
            a1v = a1buf[pl.ds(j * 16, 16)]
            dv = dbuf[pl.ds(j * 16, 16)]
            e0 = jnp.where(valid, jnp.exp(a0v - msplat), 0.0)
            e1 = jnp.where(valid, jnp.exp(a1v - msplat), 0.0)
            e0buf[pl.ds(j * 16, 16)] = e0
            e1buf[pl.ds(j * 16, 16)] = e1
            dbuf[pl.ds(j * 16, 16)] = jnp.where(valid, dv, 0)
        pltpu.sync_copy(e0buf, d0s.at[dbuf], add=True)
        pltpu.sync_copy(e1buf, d1s.at[dbuf], add=True)

    def body(i, carry):
        do_window(base0 + i * W, W)
        return carry

    lax.fori_loop(0, FW, body, 0)
    do_window(base0 + FW * W, REM)

    plsc.subcore_barrier()
    pltpu.sync_copy(d0s.at[pl.ds(sid * DSL, DSL)], zbuf)
    pltpu.sync_copy(zbuf, dp_o.at[pl.ds((cid * 2) * DPAD + sid * DSL, DSL)])
    pltpu.sync_copy(d1s.at[pl.ds(sid * DSL, DSL)], zbuf)
    pltpu.sync_copy(zbuf, dp_o.at[pl.ds((cid * 2 + 1) * DPAD + sid * DSL, DSL)])


@functools.partial(
    pl.kernel,
    out_type=(
        jax.ShapeDtypeStruct((E,), _f32),
        jax.ShapeDtypeStruct((E,), _f32),
    ),
    mesh=_mesh,
    scratch_types=[
        pltpu.VMEM((W,), _i32),
        pltpu.VMEM((W,), _f32),
        pltpu.VMEM((W,), _f32),
        pltpu.SemaphoreType.DMA,
    ],
)
def _sc_gather2(d0h, d1h, dsth, dn0_o, dn1_o, dbuf, g0buf, g1buf, sem):
    wid = lax.axis_index("c") * NS + lax.axis_index("s")
    base0 = wid * EPW

    def do_window(base, n):
        pltpu.sync_copy(dsth.at[pl.ds(base, n)], dbuf.at[pl.ds(0, n)])
        c1 = pltpu.async_copy(d0h.at[dbuf], g0buf, sem)
        c2 = pltpu.async_copy(d1h.at[dbuf], g1buf, sem)
        c1.wait(); c2.wait()
        pltpu.sync_copy(g0buf.at[pl.ds(0, n)], dn0_o.at[pl.ds(base, n)])
        pltpu.sync_copy(g1buf.at[pl.ds(0, n)], dn1_o.at[pl.ds(base, n)])

    def body(i, carry):
        do_window(base0 + i * W, W)
        return carry

    lax.fori_loop(0, FW, body, 0)
    zero = jnp.zeros((16,), _i32)
    for j in range(W // 16):
        dbuf[pl.ds(j * 16, 16)] = zero
    do_window(base0 + FW * W, REM)


def _wv_body(a0_ref, a1_ref, dn0_ref, dn1_ref, vs_ref, e_ref, m_ref, o_ref):
    mval = m_ref[0, 0]
    attn0 = jnp.exp(a0_ref[0, 0, :] - mval) / (dn0_ref[0, 0, :] + 1e-16)
    attn1 = jnp.exp(a1_ref[0, 0, :] - mval) / (dn1_ref[0, 0, :] + 1e-16)
    col = lax.broadcasted_iota(_i32, (BE, D), 1)
    att = jnp.where(col < C, attn0.reshape(BE, 1), attn1.reshape(BE, 1))
    wv = att * (vs_ref[:, 0:D] + e_ref[...])
    o_ref[...] = jnp.concatenate([wv, jnp.zeros((BE, DP - D), _f32)], axis=1)


def _wv(a0, a1, dn0, dn1, vs, e, m2d):
    nb = E // BE
    tri = lambda i: (i, 0, 0)
    return pl.pallas_call(
        _wv_body,
        grid=(nb,),
        in_specs=[
            pl.BlockSpec((1, 1, BE), tri),
            pl.BlockSpec((1, 1, BE), tri),
            pl.BlockSpec((1, 1, BE), tri),
            pl.BlockSpec((1, 1, BE), tri),
            pl.BlockSpec((BE, DP), lambda i: (i, 0)),
            pl.BlockSpec((BE, D), lambda i: (i, 0)),
            pl.BlockSpec((1, 1), lambda i: (0, 0)),
        ],
        out_specs=pl.BlockSpec((BE, DP), lambda i: (i, 0)),
        out_shape=jax.ShapeDtypeStruct((E, DP), _f32),
    )(a0, a1, dn0.reshape(nb, 1, BE), dn1.reshape(nb, 1, BE), vs, e, m2d)


@functools.partial(
    pl.kernel,
    out_type=jax.ShapeDtypeStruct((N, DP), _f32),
    mesh=_mesh,
    scratch_types=[
        pltpu.VMEM((W,), _i32),
        pltpu.VMEM((W,), _i32),
        pltpu.VMEM((W, DP), _f32),
        pltpu.VMEM_SHARED((RNG + NJUNK, DP), _f32),
    ],
)
def _sc_segsum(dsth, wvh, skiph, out_o, dbuf, libuf, wvbuf, slab):
    cid = lax.axis_index("c")
    sid = lax.axis_index("s")
    base0 = sid * EPT3
    iota = _iota16()

    for p in range(2):
        r0 = (2 * p) * RNG + cid * RNG
        t15 = (744, N - 3 * RNG - 15 * 784)

        @pl.when(sid < NS - 1)
        def _():
            pltpu.sync_copy(skiph.at[pl.ds(r0 + sid * 784, 784)],
                            slab.at[pl.ds(sid * 784, 784)])

        @pl.when((sid == NS - 1) & ((p == 0) | (cid == 0)))
        def _():
            pltpu.sync_copy(skiph.at[pl.ds(r0 + 15 * 784, t15[0])],
                            slab.at[pl.ds(15 * 784, t15[0])])

        @pl.when((sid == NS - 1) & ((p == 1) & (cid == 1)))
        def _():
            pltpu.sync_copy(skiph.at[pl.ds(r0 + 15 * 784, t15[1])],
                            slab.at[pl.ds(15 * 784, t15[1])])

        plsc.subcore_barrier()

        def do_window(base, n):
            pltpu.sync_copy(dsth.at[pl.ds(base, n)], dbuf.at[pl.ds(0, n)])
            pltpu.sync_copy(wvh.at[pl.ds(base, n)], wvbuf.at[pl.ds(0, n)])
            for j in range(W // 16):
                valid = (j * 16 + iota) < n
                dv = dbuf[pl.ds(j * 16, 16)]
                dl = dv - r0
                inb = valid & (dl >= 0) & (dl < RNG)
                junk = RNG + ((j * 16 + iota) & (NJUNK - 1))
                libuf[pl.ds(j * 16, 16)] = jnp.where(inb, dl, junk)
            pltpu.sync_copy(wvbuf, slab.at[libuf], add=True)

        def body(i, carry):
            do_window(base0 + i * W, W)
            return carry

        lax.fori_loop(0, FW3, body, 0)
        do_window(base0 + FW3 * W, REM3)

        plsc.subcore_barrier()

        @pl.when(sid < NS - 1)
        def _():
            pltpu.sync_copy(slab.at[pl.ds(sid * 784, 784)],
                            out_o.at[pl.ds(r0 + sid * 784, 784)])

        @pl.when((sid == NS - 1) & ((p == 0) | (cid == 0)))
        def _():
            pltpu.sync_copy(slab.at[pl.ds(15 * 784, t15[0])],
                            out_o.at[pl.ds(r0 + 15 * 784, t15[0])])

        @pl.when((sid == NS - 1) & ((p == 1) & (cid == 1)))
        def _():
            pltpu.sync_copy(slab.at[pl.ds(15 * 784, t15[1])],
                            out_o.at[pl.ds(r0 + 15 * 784, t15[1])])

        plsc.subcore_barrier()


def kernel(x, last_update, edge_index, t, msg, Wq, bq, Wk, bk, Wv, bv, We, Wskip, bskip, Wt, bt):
    src = edge_index[0].astype(_i32)
    dst = edge_index[1].astype(_i32)

    zw = jnp.zeros((D, DP - D), _f32)
    zb = jnp.zeros((DP - D,), _f32)
    W4 = jnp.concatenate([Wq, zw, Wk, zw, Wv, zw, Wskip, zw], axis=1)
    b4 = jnp.concatenate([bq, zb, bk, zb, bv, zb, bskip, zb]).reshape(1, 4 * DP)
    q, k, v, skip = _project_qkvs(x, W4, b4)

    lu_src, qd, ks, vs = _sc_gather1(last_update, q, k, v, src, dst)
    e, a0, a1, mpart = _edge_alpha(lu_src, t, msg, qd, ks, Wt, bt, We)

    mval = jnp.max(mpart)
    dpart = _sc_denom(a0.reshape(E), a1.reshape(E), dst,
                      jnp.full((16,), mval, _f32))
    dpart = dpart.reshape(NC * H, DPAD)
    d0 = dpart[0] + dpart[2]
    d1 = dpart[1] + dpart[3]
    dn0, dn1 = _sc_gather2(d0, d1, dst)

    wv = _wv(a0, a1, dn0, dn1, vs, e, mval.reshape(1, 1))
    out = _sc_segsum(dst, wv, skip)
    return out[:, 0:D]


# speedup vs baseline: 15.5291x; 1.0136x over previous
import functools
import jax
import jax.numpy as jnp
from jax import lax
from jax.experimental import pallas as pl
from jax.experimental.pallas import tpu as pltpu
from jax.experimental.pallas import tpu_sc as plsc

N = 50000
E = 800000
D = 100
MSG = 100
TDIM = 100
H = 2
C = D // H
EDGE_DIM = MSG + TDIM

BN = 2000
BE = 6400

NC = 2
NS = 16
NW = NC * NS
EPW = E // NW
W = 128
FW = EPW // W
REM = EPW - FW * W

EPT3 = E // NS
FW3 = EPT3 // W
REM3 = EPT3 - FW3 * W
RNG = 12504
NJUNK = 64
DPAD = 50176
DSL = DPAD // NS

_mesh = plsc.VectorSubcoreMesh(
    core_axis_name="c", subcore_axis_name="s", num_cores=NC, num_subcores=NS)

_f32 = jnp.float32
_i32 = jnp.int32


def _iota16():
    return lax.iota(_i32, 16)


DP = 128


def _qkvs_body(x_ref, w_ref, b_ref, o_q, o_k, o_v, o_s):
    r = jnp.dot(x_ref[...], w_ref[...], preferred_element_type=_f32) + b_ref[...]
    o_q[...] = r[:, 0:DP]
    o_k[...] = r[:, DP:2 * DP]
    o_v[...] = r[:, 2 * DP:3 * DP]
    o_s[...] = r[:, 3 * DP:3 * DP + D]


def _project_qkvs(x, W4, b4):
    return pl.pallas_call(
        _qkvs_body,
        grid=(N // BN,),
        in_specs=[
            pl.BlockSpec((BN, D), lambda i: (i, 0)),
            pl.BlockSpec((D, 4 * DP), lambda i: (0, 0)),
            pl.BlockSpec((1, 4 * DP), lambda i: (0, 0)),
        ],
        out_specs=[pl.BlockSpec((BN, DP), lambda i: (i, 0))] * 3
        + [pl.BlockSpec((BN, D), lambda i: (i, 0))],
        out_shape=[jax.ShapeDtypeStruct((N, DP), _f32)] * 3
        + [jax.ShapeDtypeStruct((N, D), _f32)],
    )(x, W4, b4)


@functools.partial(
    pl.kernel,
    out_type=(
        jax.ShapeDtypeStruct((E,), _f32),
        jax.ShapeDtypeStruct((E, DP), _f32),
        jax.ShapeDtypeStruct((E, DP), _f32),
        jax.ShapeDtypeStruct((E, DP), _f32),
    ),
    mesh=_mesh,
    scratch_types=[
        pltpu.VMEM((W,), _i32),
        pltpu.VMEM((W,), _i32),
        pltpu.VMEM((W,), _f32),
        pltpu.VMEM((W, DP), _f32),
        pltpu.VMEM((W, DP), _f32),
        pltpu.VMEM((W, DP), _f32),
        pltpu.SemaphoreType.DMA,
    ],
)
def _sc_gather1(lastu, qh, kh, vh, srch, dsth, lu_o, qd_o, ks_o, vs_o,
                sbuf, dbuf, lubuf, qbuf, kbuf, vbuf, sem):
    wid = lax.axis_index("c") * NS + lax.axis_index("s")
    base0 = wid * EPW

    def do_window(base, n):
        pltpu.sync_copy(srch.at[pl.ds(base, n)], sbuf.at[pl.ds(0, n)])
        pltpu.sync_copy(dsth.at[pl.ds(base, n)], dbuf.at[pl.ds(0, n)])
        c1 = pltpu.async_copy(lastu.at[sbuf], lubuf, sem)
        c2 = pltpu.async_copy(qh.at[dbuf], qbuf, sem)
        c3 = pltpu.async_copy(kh.at[sbuf], kbuf, sem)
        c4 = pltpu.async_copy(vh.at[sbuf], vbuf, sem)
        c1.wait(); c2.wait(); c3.wait(); c4.wait()
        pltpu.sync_copy(lubuf.at[pl.ds(0, n)], lu_o.at[pl.ds(base, n)])
        pltpu.sync_copy(qbuf.at[pl.ds(0, n)], qd_o.at[pl.ds(base, n)])
        pltpu.sync_copy(kbuf.at[pl.ds(0, n)], ks_o.at[pl.ds(base, n)])
        pltpu.sync_copy(vbuf.at[pl.ds(0, n)], vs_o.at[pl.ds(base, n)])

    def body(i, carry):
        do_window(base0 + i * W, W)
        return carry

    lax.fori_loop(0, FW, body, 0)
    zero = jnp.zeros((16,), _i32)
    for j in range(W // 16):
        sbuf[pl.ds(j * 16, 16)] = zero
        dbuf[pl.ds(j * 16, 16)] = zero
    do_window(base0 + FW * W, REM)


def _edge_body(lu_ref, t_ref, msg_ref, qd_ref, ks_ref, wt_ref, bt_ref, we_ref,
               e_ref, a0_ref, a1_ref, m_ref):
    i = pl.program_id(0)
    rel = lu_ref[...] - t_ref[...]
    enc = jnp.cos(rel.reshape(BE, 1) * wt_ref[...] + bt_ref[...])
    ea = jnp.concatenate([enc, msg_ref[...]], axis=1)
    e = jnp.dot(ea, we_ref[...], preferred_element_type=_f32)
    e_ref[...] = e
    prod = qd_ref[:, 0:D] * (ks_ref[:, 0:D] + e)
    col = lax.broadcasted_iota(_i32, (BE, D), 1)
    inv = 1.0 / jnp.sqrt(jnp.float32(C))
    a0 = jnp.sum(jnp.where(col < C, prod, 0.0), axis=1) * inv
    a1 = jnp.sum(jnp.where(col >= C, prod, 0.0), axis=1) * inv
    a0_ref[...] = a0.reshape(1, 1, BE)
    a1_ref[...] = a1.reshape(1, 1, BE)
    bm = jnp.maximum(jnp.max(a0), jnp.max(a1))

    @pl.when(i == 0)
    def _():
        m_ref[...] = jnp.full((8, 128), -jnp.inf, _f32)

    m_ref[...] = jnp.maximum(m_ref[...], bm)


def _edge_alpha(lu_src, t, msg, qd, ks, Wt, bt, We):
    nb = E // BE
    return pl.pallas_call(
        _edge_body,
        grid=(nb,),
        in_specs=[
            pl.BlockSpec((1, 1, BE), lambda i: (i, 0, 0)),
            pl.BlockSpec((1, 1, BE), lambda i: (i, 0, 0)),
            pl.BlockSpec((BE, MSG), lambda i: (i, 0)),
            pl.BlockSpec((BE, DP), lambda i: (i, 0)),
            pl.BlockSpec((BE, DP), lambda i: (i, 0)),
            pl.BlockSpec((1, TDIM), lambda i: (0, 0)),
            pl.BlockSpec((1, TDIM), lambda i: (0, 0)),
            pl.BlockSpec((EDGE_DIM, D), lambda i: (0, 0)),
        ],
        out_specs=[
            pl.BlockSpec((BE, D), lambda i: (i, 0)),
            pl.BlockSpec((1, 1, BE), lambda i: (i, 0, 0)),
            pl.BlockSpec((1, 1, BE), lambda i: (i, 0, 0)),
            pl.BlockSpec((8, 128), lambda i: (0, 0)),
        ],
        out_shape=[
            jax.ShapeDtypeStruct((E, D), _f32),
            jax.ShapeDtypeStruct((nb, 1, BE), _f32),
            jax.ShapeDtypeStruct((nb, 1, BE), _f32),
            jax.ShapeDtypeStruct((8, 128), _f32),
        ],
    )(lu_src.reshape(nb, 1, BE), t.reshape(nb, 1, BE), msg, qd, ks,
      Wt.reshape(1, TDIM), bt.reshape(1, TDIM), We)


@functools.partial(
    pl.kernel,
    out_type=jax.ShapeDtypeStruct((NC * H * DPAD,), _f32),
    mesh=_mesh,
    scratch_types=[
        pltpu.VMEM((DSL,), _f32),
        pltpu.VMEM((W,), _i32),
        pltpu.VMEM((W,), _f32),
        pltpu.VMEM((W,), _f32),
        pltpu.VMEM((W,), _f32),
        pltpu.VMEM((W,), _f32),
        pltpu.VMEM((16,), _f32),
        pltpu.VMEM_SHARED((DPAD,), _f32),
        pltpu.VMEM_SHARED((DPAD,), _f32),
    ],
)
def _sc_denom(a0h, a1h, dsth, mh, dp_o,
              zbuf, dbuf, a0buf, a1buf, e0buf, e1buf, mbuf, d0s, d1s):
    cid = lax.axis_index("c")
    sid = lax.axis_index("s")
    wid = cid * NS + sid
    base0 = wid * EPW

    zero = jnp.zeros((16,), _f32)

    def zb(i, carry):
        zbuf[pl.ds(i * 16, 16)] = zero
        return carry

    lax.fori_loop(0, DSL // 16, zb, 0)
    pltpu.sync_copy(zbuf, d0s.at[pl.ds(sid * DSL, DSL)])
    pltpu.sync_copy(zbuf, d1s.at[pl.ds(sid * DSL, DSL)])
    pltpu.sync_copy(mh, mbuf)
    plsc.subcore_barrier()

    msplat = mbuf[...]
    iota = _iota16()

    def do_window(base, n):
        pltpu.sync_copy(dsth.at[pl.ds(base, n)], dbuf.at[pl.ds(0, n)])
        pltpu.sync_copy(a0h.at[pl.ds(base, n)], a0buf.at[pl.ds(0, n)])
        pltpu.sync_copy(a1h.at[pl.ds(base, n)], a1buf.at[pl.ds(0, n)])
        for j in range(W // 16):
            valid = (j * 16 + iota) < n
            a0v = a0buf[pl.ds(j * 16, 16)]
            a1v = a1buf[pl.ds(j * 16, 16)]
            dv = dbuf[pl.ds(j * 16, 16)]
            e0 = jnp.where(valid, jnp.exp(a0v - msplat), 0.0)
            e1 = jnp.where(valid, jnp.exp(a1v - msplat), 0.0)
            e0buf[pl.ds(j * 16, 16)] = e0
            e1buf[pl.ds(j * 16, 16)] = e1
            dbuf[pl.ds(j * 16, 16)] = jnp.where(valid, dv, 0)
        pltpu.sync_copy(e0buf, d0s.at[dbuf], add=True)
        pltpu.sync_copy(e1buf, d1s.at[dbuf], add=True)

    def body(i, carry):
        do_window(base0 + i * W, W)
        return carry

    lax.fori_loop(0, FW, body, 0)
    do_window(base0 + FW * W, REM)

    plsc.subcore_barrier()
    pltpu.sync_copy(d0s.at[pl.ds(sid * DSL, DSL)], zbuf)
    pltpu.sync_copy(zbuf, dp_o.at[pl.ds((cid * 2) * DPAD + sid * DSL, DSL)])
    pltpu.sync_copy(d1s.at[pl.ds(sid * DSL, DSL)], zbuf)
    pltpu.sync_copy(zbuf, dp_o.at[pl.ds((cid * 2 + 1) * DPAD + sid * DSL, DSL)])


@functools.partial(
    pl.kernel,
    out_type=(
        jax.ShapeDtypeStruct((E,), _f32),
        jax.ShapeDtypeStruct((E,), _f32),
    ),
    mesh=_mesh,
    scratch_types=[
        pltpu.VMEM((W,), _i32),
        pltpu.VMEM((W,), _f32),
        pltpu.VMEM((W,), _f32),
        pltpu.SemaphoreType.DMA,
    ],
)
def _sc_gather2(d0h, d1h, dsth, dn0_o, dn1_o, dbuf, g0buf, g1buf, sem):
    wid = lax.axis_index("c") * NS + lax.axis_index("s")
    base0 = wid * EPW

    def do_window(base, n):
        pltpu.sync_copy(dsth.at[pl.ds(base, n)], dbuf.at[pl.ds(0, n)])
        c1 = pltpu.async_copy(d0h.at[dbuf], g0buf, sem)
        c2 = pltpu.async_copy(d1h.at[dbuf], g1buf, sem)
        c1.wait(); c2.wait()
        pltpu.sync_copy(g0buf.at[pl.ds(0, n)], dn0_o.at[pl.ds(base, n)])
        pltpu.sync_copy(g1buf.at[pl.ds(0, n)], dn1_o.at[pl.ds(base, n)])

    def body(i, carry):
        do_window(base0 + i * W, W)
        return carry

    lax.fori_loop(0, FW, body, 0)
    zero = jnp.zeros((16,), _i32)
    for j in range(W // 16):
        dbuf[pl.ds(j * 16, 16)] = zero
    do_window(base0 + FW * W, REM)


def _wv_body(a0_ref, a1_ref, dn0_ref, dn1_ref, vs_ref, e_ref, m_ref, o_ref):
    mval = m_ref[0, 0]
    attn0 = jnp.exp(a0_ref[0, 0, :] - mval) / (dn0_ref[0, 0, :] + 1e-16)
    attn1 = jnp.exp(a1_ref[0, 0, :] - mval) / (dn1_ref[0, 0, :] + 1e-16)
    col = lax.broadcasted_iota(_i32, (BE, D), 1)
    att = jnp.where(col < C, attn0.reshape(BE, 1), attn1.reshape(BE, 1))
    o_ref[...] = att * (vs_ref[:, 0:D] + e_ref[...])


def _wv(a0, a1, dn0, dn1, vs, e, m2d):
    nb = E // BE
    tri = lambda i: (i, 0, 0)
    return pl.pallas_call(
        _wv_body,
        grid=(nb,),
        in_specs=[
            pl.BlockSpec((1, 1, BE), tri),
            pl.BlockSpec((1, 1, BE), tri),
            pl.BlockSpec((1, 1, BE), tri),
            pl.BlockSpec((1, 1, BE), tri),
            pl.BlockSpec((BE, DP), lambda i: (i, 0)),
            pl.BlockSpec((BE, D), lambda i: (i, 0)),
            pl.BlockSpec((1, 1), lambda i: (0, 0)),
        ],
        out_specs=pl.BlockSpec((BE, D), lambda i: (i, 0)),
        out_shape=jax.ShapeDtypeStruct((E, D), _f32),
    )(a0, a1, dn0.reshape(nb, 1, BE), dn1.reshape(nb, 1, BE), vs, e, m2d)


@functools.partial(
    pl.kernel,
    out_type=jax.ShapeDtypeStruct((N, D), _f32),
    mesh=_mesh,
    scratch_types=[
        pltpu.VMEM((W,), _i32),
        pltpu.VMEM((W,), _i32),
        pltpu.VMEM((W, D), _f32),
        pltpu.VMEM_SHARED((RNG + NJUNK, D), _f32),
    ],
)
def _sc_segsum(dsth, wvh, skiph, out_o, dbuf, libuf, wvbuf, slab):
    cid = lax.axis_index("c")
    sid = lax.axis_index("s")
    base0 = sid * EPT3
    iota = _iota16()

    for p in range(2):
        r0 = (2 * p) * RNG + cid * RNG
        t15 = (744, N - 3 * RNG - 15 * 784)

        @pl.when(sid < NS - 1)
        def _():
            pltpu.sync_copy(skiph.at[pl.ds(r0 + sid * 784, 784)],
                            slab.at[pl.ds(sid * 784, 784)])

        @pl.when((sid == NS - 1) & ((p == 0) | (cid == 0)))
        def _():
            pltpu.sync_copy(skiph.at[pl.ds(r0 + 15 * 784, t15[0])],
                            slab.at[pl.ds(15 * 784, t15[0])])

        @pl.when((sid == NS - 1) & ((p == 1) & (cid == 1)))
        def _():
            pltpu.sync_copy(skiph.at[pl.ds(r0 + 15 * 784, t15[1])],
                            slab.at[pl.ds(15 * 784, t15[1])])

        plsc.subcore_barrier()

        def do_window(base, n):
            pltpu.sync_copy(dsth.at[pl.ds(base, n)], dbuf.at[pl.ds(0, n)])
            pltpu.sync_copy(wvh.at[pl.ds(base, n)], wvbuf.at[pl.ds(0, n)])
            for j in range(W // 16):
                valid = (j * 16 + iota) < n
                dv = dbuf[pl.ds(j * 16, 16)]
                dl = dv - r0
                inb = valid & (dl >= 0) & (dl < RNG)
                junk = RNG + ((j * 16 + iota) & (NJUNK - 1))
                libuf[pl.ds(j * 16, 16)] = jnp.where(inb, dl, junk)
            pltpu.sync_copy(wvbuf, slab.at[libuf], add=True)

        def body(i, carry):
            do_window(base0 + i * W, W)
            return carry

        lax.fori_loop(0, FW3, body, 0)
        do_window(base0 + FW3 * W, REM3)

        plsc.subcore_barrier()

        @pl.when(sid < NS - 1)
        def _():
            pltpu.sync_copy(slab.at[pl.ds(sid * 784, 784)],
                            out_o.at[pl.ds(r0 + sid * 784, 784)])

        @pl.when((sid == NS - 1) & ((p == 0) | (cid == 0)))
        def _():
            pltpu.sync_copy(slab.at[pl.ds(15 * 784, t15[0])],
                            out_o.at[pl.ds(r0 + 15 * 784, t15[0])])

        @pl.when((sid == NS - 1) & ((p == 1) & (cid == 1)))
        def _():
            pltpu.sync_copy(slab.at[pl.ds(15 * 784, t15[1])],
                            out_o.at[pl.ds(r0 + 15 * 784, t15[1])])

        plsc.subcore_barrier()


def kernel(x, last_update, edge_index, t, msg, Wq, bq, Wk, bk, Wv, bv, We, Wskip, bskip, Wt, bt):
    src = edge_index[0].astype(_i32)
    dst = edge_index[1].astype(_i32)

    zw = jnp.zeros((D, DP - D), _f32)
    zb = jnp.zeros((DP - D,), _f32)
    W4 = jnp.concatenate([Wq, zw, Wk, zw, Wv, zw, Wskip, zw], axis=1)
    b4 = jnp.concatenate([bq, zb, bk, zb, bv, zb, bskip, zb]).reshape(1, 4 * DP)
    q, k, v, skip = _project_qkvs(x, W4, b4)

    lu_src, qd, ks, vs = _sc_gather1(last_update, q, k, v, src, dst)
    e, a0, a1, mpart = _edge_alpha(lu_src, t, msg, qd, ks, Wt, bt, We)

    mval = jnp.max(mpart)
    dpart = _sc_denom(a0.reshape(E), a1.reshape(E), dst,
                      jnp.full((16,), mval, _f32))
    dpart = dpart.reshape(NC * H, DPAD)
    d0 = dpart[0] + dpart[2]
    d1 = dpart[1] + dpart[3]
    dn0, dn1 = _sc_gather2(d0, d1, dst)

    wv = _wv(a0, a1, dn0, dn1, vs, e, mval.reshape(1, 1))
    return _sc_segsum(dst, wv, skip)
